# baseline (speedup 1.0000x reference)
import functools

import jax
import jax.numpy as jnp
from jax import lax
from jax.experimental import pallas as pl
from jax.experimental.pallas import tpu as pltpu
from jax.experimental.pallas import tpu_sc as plsc

V = 100000
D = 128
H = 256
B = 1024
T = 50
G = 4 * H

_NW = 32
_CHUNK = 80


def _sc_gather(table, idx):
    n = idx.shape[0]
    bpw = n // _NW
    nch = bpw // _CHUNK
    idx3 = idx.reshape(_NW, nch, _CHUNK)
    mesh = plsc.VectorSubcoreMesh(core_axis_name="c", subcore_axis_name="s")

    @functools.partial(
        pl.kernel,
        out_type=jax.ShapeDtypeStruct((n, D), jnp.float32),
        mesh=mesh,
        scratch_types=[
            pltpu.VMEM((nch, _CHUNK), jnp.int32),
            pltpu.VMEM((_CHUNK, D), jnp.float32),
            pltpu.VMEM((_CHUNK, D), jnp.float32),
            pltpu.SemaphoreType.DMA,
            pltpu.SemaphoreType.DMA,
        ],
    )
    def gather_k(table_hbm, idx_hbm, out_hbm, idx_v, rows0, rows1, sem0, sem1):
        wid = lax.axis_index("s") * 2 + lax.axis_index("c")
        base = wid * bpw
        pltpu.sync_copy(idx_hbm.at[wid], idx_v)
        rows = (rows0, rows1)
        sems = (sem0, sem1)
        pending = pltpu.async_copy(table_hbm.at[idx_v.at[0]], rows0, sem0)
        for ci in range(nch):
            pending.wait()
            if ci + 1 < nch:
                nxt = pltpu.async_copy(
                    table_hbm.at[idx_v.at[ci + 1]], rows[(ci + 1) % 2],
                    sems[(ci + 1) % 2])
            pltpu.sync_copy(rows[ci % 2],
                            out_hbm.at[pl.ds(base + ci * _CHUNK, _CHUNK)])
            if ci + 1 < nch:
                pending = nxt

    return gather_k(table, idx3)


def _dotT(a, w):
    return lax.dot_general(a.astype(jnp.bfloat16), w, (((1,), (1,)), ((), ())),
                           preferred_element_type=jnp.float32)


def _sig(v):
    return 0.5 * jnp.tanh(0.5 * v) + 0.5


def _scan_body(xf_ref, xb_ref, wf_ref, bf_ref, wb_ref, bb_ref,
               hsf_ref, hsb_ref, pool_ref, hf, cf, hb, cb):
    t = pl.program_id(0)

    @pl.when(t == 0)
    def _init():
        hf[...] = jnp.zeros_like(hf)
        cf[...] = jnp.zeros_like(cf)
        hb[...] = jnp.zeros_like(hb)
        cb[...] = jnp.zeros_like(cb)
        pool_ref[...] = jnp.zeros_like(pool_ref)

    def cell(x, h, c, w, bias):
        xh = jnp.concatenate([x, h], axis=1)
        gates = _dotT(xh, w).astype(jnp.bfloat16) + bias
        i = _sig(gates[:, 0 * H:1 * H])
        f = _sig(gates[:, 1 * H:2 * H])
        g = jnp.tanh(gates[:, 2 * H:3 * H])
        o = _sig(gates[:, 3 * H:4 * H])
        c_new = f.astype(jnp.float32) * c + (i * g).astype(jnp.float32)
        h_new = o * jnp.tanh(c_new).astype(jnp.bfloat16)
        return h_new, c_new

    h_f, c_f = cell(xf_ref[0], hf[...], cf[...], wf_ref[...], bf_ref[...])
    hf[...] = h_f
    cf[...] = c_f
    hsf_ref[0] = h_f

    h_b, c_b = cell(xb_ref[0], hb[...], cb[...], wb_ref[...], bb_ref[...])
    hb[...] = h_b
    cb[...] = c_b
    hsb_ref[0] = h_b

    pool_ref[:, :H] += h_f.astype(jnp.float32) * (1.0 / T)
    pool_ref[:, H:] += h_b.astype(jnp.float32) * (1.0 / T)


def _bilstm_scan(emb3, wf, bf, wb, bb):
    const = lambda *_: (0, 0)
    return pl.pallas_call(
        _scan_body,
        grid=(T,),
        in_specs=[
            pl.BlockSpec((1, B, D), lambda t: (t, 0, 0)),
            pl.BlockSpec((1, B, D), lambda t: (T - 1 - t, 0, 0)),
            pl.BlockSpec((G, D + H), const),
            pl.BlockSpec((1, G), const),
            pl.BlockSpec((G, D + H), const),
            pl.BlockSpec((1, G), const),
        ],
        out_specs=[
            pl.BlockSpec((1, B, H), lambda t: (t, 0, 0)),
            pl.BlockSpec((1, B, H), lambda t: (T - 1 - t, 0, 0)),
            pl.BlockSpec((B, 2 * H), const),
        ],
        out_shape=[
            jax.ShapeDtypeStruct((T, B, H), jnp.bfloat16),
            jax.ShapeDtypeStruct((T, B, H), jnp.bfloat16),
            jax.ShapeDtypeStruct((B, 2 * H), jnp.float32),
        ],
        scratch_shapes=[
            pltpu.VMEM((B, H), jnp.bfloat16),
            pltpu.VMEM((B, H), jnp.float32),
            pltpu.VMEM((B, H), jnp.bfloat16),
            pltpu.VMEM((B, H), jnp.float32),
        ],
    )(emb3, emb3, wf, bf, wb, bb)


def kernel(title_text_token_ids, embedding_weight, W_ih_f, W_hh_f, b_ih_f,
           b_hh_f, W_ih_b, W_hh_b, b_ih_b, b_hh_b):
    idx = title_text_token_ids.astype(jnp.int32).T.reshape(-1)
    emb = _sc_gather(embedding_weight, idx)
    emb3 = emb.reshape(T, B, D).astype(jnp.bfloat16)
    bf = (b_ih_f + b_hh_f).reshape(1, G).astype(jnp.bfloat16)
    bb = (b_ih_b + b_hh_b).reshape(1, G).astype(jnp.bfloat16)
    wf = jnp.concatenate([W_ih_f, W_hh_f], axis=1).astype(jnp.bfloat16)
    wb = jnp.concatenate([W_ih_b, W_hh_b], axis=1).astype(jnp.bfloat16)
    hs_f, hs_b, pool = _bilstm_scan(emb3, wf, bf, wb, bb)
    lstm_out = jnp.concatenate(
        [hs_f.transpose(1, 0, 2), hs_b.transpose(1, 0, 2)],
        axis=-1).astype(jnp.float32)
    return (lstm_out, pool)


# speedup vs baseline: 1.0309x; 1.0309x over previous
import functools

import jax
import jax.numpy as jnp
from jax import lax
from jax.experimental import pallas as pl
from jax.experimental.pallas import tpu as pltpu
from jax.experimental.pallas import tpu_sc as plsc

V = 100000
D = 128
H = 256
B = 1024
T = 50
G = 4 * H

_NW = 32
_CHUNK = 80


def _sc_gather(table, idx):
    n = idx.shape[0]
    bpw = n // _NW
    nch = bpw // _CHUNK
    idx3 = idx.reshape(_NW, nch, _CHUNK)
    mesh = plsc.VectorSubcoreMesh(core_axis_name="c", subcore_axis_name="s")

    @functools.partial(
        pl.kernel,
        out_type=jax.ShapeDtypeStruct((n, D), jnp.float32),
        mesh=mesh,
        scratch_types=[
            pltpu.VMEM((nch, _CHUNK), jnp.int32),
            pltpu.VMEM((_CHUNK, D), jnp.float32),
            pltpu.VMEM((_CHUNK, D), jnp.float32),
            pltpu.SemaphoreType.DMA,
            pltpu.SemaphoreType.DMA,
        ],
    )
    def gather_k(table_hbm, idx_hbm, out_hbm, idx_v, rows0, rows1, sem0, sem1):
        wid = lax.axis_index("s") * 2 + lax.axis_index("c")
        base = wid * bpw
        pltpu.sync_copy(idx_hbm.at[wid], idx_v)
        rows = (rows0, rows1)
        sems = (sem0, sem1)
        pending = pltpu.async_copy(table_hbm.at[idx_v.at[0]], rows0, sem0)
        for ci in range(nch):
            pending.wait()
            if ci + 1 < nch:
                nxt = pltpu.async_copy(
                    table_hbm.at[idx_v.at[ci + 1]], rows[(ci + 1) % 2],
                    sems[(ci + 1) % 2])
            pltpu.sync_copy(rows[ci % 2],
                            out_hbm.at[pl.ds(base + ci * _CHUNK, _CHUNK)])
            if ci + 1 < nch:
                pending = nxt

    return gather_k(table, idx3)


def _dotT(a, w):
    return lax.dot_general(a.astype(jnp.bfloat16), w, (((1,), (1,)), ((), ())),
                           preferred_element_type=jnp.float32)


def _sig(v):
    return 0.5 * jnp.tanh(0.5 * v) + 0.5


def _scan_body(xf_ref, xb_ref, wf_ref, bf_ref, wb_ref, bb_ref,
               hsf_ref, hsb_ref, pool_ref, hf, cf, hb, cb):
    t = pl.program_id(0)

    @pl.when(t == 0)
    def _init():
        hf[...] = jnp.zeros_like(hf)
        cf[...] = jnp.zeros_like(cf)
        hb[...] = jnp.zeros_like(hb)
        cb[...] = jnp.zeros_like(cb)
        pool_ref[...] = jnp.zeros_like(pool_ref)

    def cell(x, h, c, w, bias):
        xh = jnp.concatenate([x, h], axis=1)
        gates = _dotT(xh, w).astype(jnp.bfloat16) + bias
        i = _sig(gates[:, 0 * H:1 * H])
        f = _sig(gates[:, 1 * H:2 * H])
        g = jnp.tanh(gates[:, 2 * H:3 * H])
        o = _sig(gates[:, 3 * H:4 * H])
        c_new = f.astype(jnp.float32) * c + (i * g).astype(jnp.float32)
        h_new = o * jnp.tanh(c_new).astype(jnp.bfloat16)
        return h_new, c_new

    HB = B // 2
    for lo in (0, HB):
        rows = pl.ds(lo, HB)
        h_f, c_f = cell(xf_ref[0, rows], hf[rows], cf[rows], wf_ref[...],
                        bf_ref[...])
        hf[rows] = h_f
        cf[rows] = c_f
        hsf_ref[0, rows] = h_f

        h_b, c_b = cell(xb_ref[0, rows], hb[rows], cb[rows], wb_ref[...],
                        bb_ref[...])
        hb[rows] = h_b
        cb[rows] = c_b
        hsb_ref[0, rows] = h_b

        pool_ref[rows, :H] += h_f.astype(jnp.float32) * (1.0 / T)
        pool_ref[rows, H:] += h_b.astype(jnp.float32) * (1.0 / T)


def _bilstm_scan(emb3, wf, bf, wb, bb):
    const = lambda *_: (0, 0)
    return pl.pallas_call(
        _scan_body,
        grid=(T,),
        in_specs=[
            pl.BlockSpec((1, B, D), lambda t: (t, 0, 0)),
            pl.BlockSpec((1, B, D), lambda t: (T - 1 - t, 0, 0)),
            pl.BlockSpec((G, D + H), const),
            pl.BlockSpec((1, G), const),
            pl.BlockSpec((G, D + H), const),
            pl.BlockSpec((1, G), const),
        ],
        out_specs=[
            pl.BlockSpec((1, B, H), lambda t: (t, 0, 0)),
            pl.BlockSpec((1, B, H), lambda t: (T - 1 - t, 0, 0)),
            pl.BlockSpec((B, 2 * H), const),
        ],
        out_shape=[
            jax.ShapeDtypeStruct((T, B, H), jnp.bfloat16),
            jax.ShapeDtypeStruct((T, B, H), jnp.bfloat16),
            jax.ShapeDtypeStruct((B, 2 * H), jnp.float32),
        ],
        scratch_shapes=[
            pltpu.VMEM((B, H), jnp.bfloat16),
            pltpu.VMEM((B, H), jnp.float32),
            pltpu.VMEM((B, H), jnp.bfloat16),
            pltpu.VMEM((B, H), jnp.float32),
        ],
    )(emb3, emb3, wf, bf, wb, bb)


def kernel(title_text_token_ids, embedding_weight, W_ih_f, W_hh_f, b_ih_f,
           b_hh_f, W_ih_b, W_hh_b, b_ih_b, b_hh_b):
    idx = title_text_token_ids.astype(jnp.int32).T.reshape(-1)
    emb = _sc_gather(embedding_weight, idx)
    emb3 = emb.reshape(T, B, D).astype(jnp.bfloat16)
    bf = (b_ih_f + b_hh_f).reshape(1, G).astype(jnp.bfloat16)
    bb = (b_ih_b + b_hh_b).reshape(1, G).astype(jnp.bfloat16)
    wf = jnp.concatenate([W_ih_f, W_hh_f], axis=1).astype(jnp.bfloat16)
    wb = jnp.concatenate([W_ih_b, W_hh_b], axis=1).astype(jnp.bfloat16)
    hs_f, hs_b, pool = _bilstm_scan(emb3, wf, bf, wb, bb)
    lstm_out = jnp.concatenate(
        [hs_f.transpose(1, 0, 2), hs_b.transpose(1, 0, 2)],
        axis=-1).astype(jnp.float32)
    return (lstm_out, pool)


# speedup vs baseline: 1.1009x; 1.0680x over previous
import functools

import jax
import jax.numpy as jnp
from jax import lax
from jax.experimental import pallas as pl
from jax.experimental.pallas import tpu as pltpu
from jax.experimental.pallas import tpu_sc as plsc

V = 100000
D = 128
H = 256
B = 1024
T = 50
G = 4 * H

_NW = 32
_CHUNK = 80


def _sc_gather(table, idx):
    n = idx.shape[0]
    bpw = n // _NW
    nch = bpw // _CHUNK
    idx3 = idx.reshape(_NW, nch, _CHUNK)
    mesh = plsc.VectorSubcoreMesh(core_axis_name="c", subcore_axis_name="s")

    @functools.partial(
        pl.kernel,
        out_type=jax.ShapeDtypeStruct((n, D), jnp.float32),
        mesh=mesh,
        scratch_types=[
            pltpu.VMEM((nch, _CHUNK), jnp.int32),
            pltpu.VMEM((_CHUNK, D), jnp.float32),
            pltpu.VMEM((_CHUNK, D), jnp.float32),
            pltpu.SemaphoreType.DMA,
            pltpu.SemaphoreType.DMA,
        ],
    )
    def gather_k(table_hbm, idx_hbm, out_hbm, idx_v, rows0, rows1, sem0, sem1):
        wid = lax.axis_index("s") * 2 + lax.axis_index("c")
        base = wid * bpw
        pltpu.sync_copy(idx_hbm.at[wid], idx_v)
        rows = (rows0, rows1)
        sems = (sem0, sem1)
        pending = pltpu.async_copy(table_hbm.at[idx_v.at[0]], rows0, sem0)
        for ci in range(nch):
            pending.wait()
            if ci + 1 < nch:
                nxt = pltpu.async_copy(
                    table_hbm.at[idx_v.at[ci + 1]], rows[(ci + 1) % 2],
                    sems[(ci + 1) % 2])
            pltpu.sync_copy(rows[ci % 2],
                            out_hbm.at[pl.ds(base + ci * _CHUNK, _CHUNK)])
            if ci + 1 < nch:
                pending = nxt

    return gather_k(table, idx3)


def _dotT(a, w):
    return lax.dot_general(a.astype(jnp.bfloat16), w, (((1,), (1,)), ((), ())),
                           preferred_element_type=jnp.float32)


def _sig(v):
    return 0.5 * jnp.tanh(0.5 * v) + 0.5


def _scan_body(xf_ref, xb_ref, wf_ref, bf_ref, wb_ref, bb_ref,
               hsf_ref, hsb_ref, pool_ref, hf, cf, hb, cb):
    t = pl.program_id(0)

    @pl.when(t == 0)
    def _init():
        hf[...] = jnp.zeros_like(hf)
        cf[...] = jnp.zeros_like(cf)
        hb[...] = jnp.zeros_like(hb)
        cb[...] = jnp.zeros_like(cb)
        pool_ref[...] = jnp.zeros_like(pool_ref)

    def cell(x, h, c, w, bias):
        xh = jnp.concatenate([x.astype(jnp.bfloat16), h], axis=1)
        gates = _dotT(xh, w).astype(jnp.bfloat16) + bias
        i = _sig(gates[:, 0 * H:1 * H])
        f = _sig(gates[:, 1 * H:2 * H])
        g = jnp.tanh(gates[:, 2 * H:3 * H])
        o = _sig(gates[:, 3 * H:4 * H])
        c_new = f.astype(jnp.float32) * c + (i * g).astype(jnp.float32)
        h_new = o * jnp.tanh(c_new).astype(jnp.bfloat16)
        return h_new, c_new

    HB = B // 4
    for lo in (0, HB, 2 * HB, 3 * HB):
        rows = pl.ds(lo, HB)
        h_f, c_f = cell(xf_ref[0, rows], hf[rows], cf[rows], wf_ref[...],
                        bf_ref[...])
        hf[rows] = h_f
        cf[rows] = c_f
        hsf_ref[0, rows] = h_f

        h_b, c_b = cell(xb_ref[0, rows], hb[rows], cb[rows], wb_ref[...],
                        bb_ref[...])
        hb[rows] = h_b
        cb[rows] = c_b
        hsb_ref[0, rows] = h_b

        pool_ref[rows, :H] += h_f.astype(jnp.float32) * (1.0 / T)
        pool_ref[rows, H:] += h_b.astype(jnp.float32) * (1.0 / T)


def _bilstm_scan(emb3, wf, bf, wb, bb):
    const = lambda *_: (0, 0)
    return pl.pallas_call(
        _scan_body,
        grid=(T,),
        in_specs=[
            pl.BlockSpec((1, B, D), lambda t: (t, 0, 0)),
            pl.BlockSpec((1, B, D), lambda t: (T - 1 - t, 0, 0)),
            pl.BlockSpec((G, D + H), const),
            pl.BlockSpec((1, G), const),
            pl.BlockSpec((G, D + H), const),
            pl.BlockSpec((1, G), const),
        ],
        out_specs=[
            pl.BlockSpec((1, B, H), lambda t: (t, 0, 0)),
            pl.BlockSpec((1, B, H), lambda t: (T - 1 - t, 0, 0)),
            pl.BlockSpec((B, 2 * H), const),
        ],
        out_shape=[
            jax.ShapeDtypeStruct((T, B, H), jnp.bfloat16),
            jax.ShapeDtypeStruct((T, B, H), jnp.bfloat16),
            jax.ShapeDtypeStruct((B, 2 * H), jnp.float32),
        ],
        scratch_shapes=[
            pltpu.VMEM((B, H), jnp.bfloat16),
            pltpu.VMEM((B, H), jnp.float32),
            pltpu.VMEM((B, H), jnp.bfloat16),
            pltpu.VMEM((B, H), jnp.float32),
        ],
    )(emb3, emb3, wf, bf, wb, bb)


def kernel(title_text_token_ids, embedding_weight, W_ih_f, W_hh_f, b_ih_f,
           b_hh_f, W_ih_b, W_hh_b, b_ih_b, b_hh_b):
    idx = title_text_token_ids.astype(jnp.int32).T.reshape(-1)
    emb = _sc_gather(embedding_weight, idx)
    emb3 = emb.reshape(T, B, D)
    bf = (b_ih_f + b_hh_f).reshape(1, G).astype(jnp.bfloat16)
    bb = (b_ih_b + b_hh_b).reshape(1, G).astype(jnp.bfloat16)
    wf = jnp.concatenate([W_ih_f, W_hh_f], axis=1).astype(jnp.bfloat16)
    wb = jnp.concatenate([W_ih_b, W_hh_b], axis=1).astype(jnp.bfloat16)
    hs_f, hs_b, pool = _bilstm_scan(emb3, wf, bf, wb, bb)
    lstm_out = jnp.concatenate(
        [hs_f.transpose(1, 0, 2), hs_b.transpose(1, 0, 2)],
        axis=-1).astype(jnp.float32)
    return (lstm_out, pool)
